# baseline (speedup 1.0000x reference)
import functools

import jax
import jax.numpy as jnp
from jax import lax
from jax.experimental import pallas as pl
from jax.experimental.pallas import tpu as pltpu
from jax.experimental.pallas import tpu_sc as plsc

_LBLK = 8192


def _transpose_pack_body(x0_ref, x1_ref, x2_ref, x3_ref, eye_ref, out_ref):
    X = jnp.concatenate(
        [x0_ref[...], x1_ref[...], x2_ref[...], x3_ref[...]], axis=0)
    out_ref[...] = lax.dot_general(
        X, eye_ref[...], (((0,), (0,)), ((), ())),
        preferred_element_type=jnp.float32)


def _make_sc_gather(B, b_per_w, NC):
    mesh = plsc.VectorSubcoreMesh(core_axis_name="c", subcore_axis_name="s")

    @functools.partial(
        pl.kernel,
        mesh=mesh,
        compiler_params=pltpu.CompilerParams(
            needs_layout_passes=False, use_tc_tiling_on_sc=True),
        out_type=jax.ShapeDtypeStruct((B, 128), jnp.float32),
        scratch_types=[
            pltpu.VMEM((b_per_w,), jnp.int32),
            pltpu.VMEM((b_per_w, 128), jnp.float32),
            pltpu.SemaphoreType.DMA,
        ],
    )
    def sc_gather(table_hbm, idx_hbm, out_hbm, idx_v, rows_v, sem):
        wid = lax.axis_index("s") * NC + lax.axis_index("c")
        base = wid * b_per_w
        pltpu.sync_copy(idx_hbm.at[pl.ds(base, b_per_w)], idx_v)
        pltpu.async_copy(table_hbm.at[idx_v], rows_v, sem).wait()
        pltpu.sync_copy(rows_v, out_hbm.at[pl.ds(base, b_per_w)])

    return sc_gather


def _content_body(x_ref, w1_ref, b1_ref, w2_ref, b2_ref, cvt_ref):
    h = jnp.maximum(
        jnp.dot(x_ref[...], w1_ref[...], preferred_element_type=jnp.float32)
        + b1_ref[...], 0.0)
    cvt_ref[...] = (
        lax.dot_general(w2_ref[...], h, (((0,), (1,)), ((), ())),
                        preferred_element_type=jnp.float32)
        + b2_ref[...])


def _final_body(mv4_ref, off_ref, cvt_ref, w3s_ref, w3b_ref, b3_ref,
                w4_ref, b4_ref, out_ref):
    seg = lax.broadcasted_iota(jnp.int32, mv4_ref.shape, 1) // 32
    mv4m = jnp.where(seg == off_ref[...], mv4_ref[...], 0.0)
    h2 = jnp.maximum(
        jnp.dot(mv4m, w3s_ref[...], preferred_element_type=jnp.float32)
        + lax.dot_general(cvt_ref[...], w3b_ref[...], (((0,), (0,)), ((), ())),
                          preferred_element_type=jnp.float32)
        + b3_ref[...], 0.0)
    out_ref[...] = (
        lax.dot_general(w4_ref[...], h2, (((0,), (1,)), ((), ())),
                        preferred_element_type=jnp.float32)
        + b4_ref[...])


def kernel(movie_ids, content_features, embed_table, W1, b1, W2, b2, W3, b3, W4, b4):
    B, NC_FEAT = content_features.shape
    V, D = embed_table.shape
    H1 = W1.shape[1]
    H2 = W2.shape[1]
    H3 = W3.shape[1]
    OUT = W4.shape[1]
    PACK = 128 // D

    info = plsc.get_sparse_core_info()
    NW = info.num_cores * info.num_subcores
    b_per_w = B // NW

    rblk = 8192
    nblk = -(-(V // PACK) // rblk)
    V4 = nblk * rblk
    ids = movie_ids.astype(jnp.int32)
    idx4 = ids % V4
    off = (ids // V4).reshape(B, 1)
    xmax = (V - 1) // rblk

    def _xmap(a):
        return lambda i: (0, jnp.minimum(i + a * nblk, xmax))

    table4 = pl.pallas_call(
        _transpose_pack_body,
        grid=(nblk,),
        in_specs=[
            pl.BlockSpec((D, rblk), _xmap(0)),
            pl.BlockSpec((D, rblk), _xmap(1)),
            pl.BlockSpec((D, rblk), _xmap(2)),
            pl.BlockSpec((D, rblk), _xmap(3)),
            pl.BlockSpec((128, 128), lambda i: (0, 0)),
        ],
        out_specs=pl.BlockSpec((rblk, 128), lambda i: (i, 0)),
        out_shape=jax.ShapeDtypeStruct((V4, 128), jnp.float32),
    )(embed_table.T, embed_table.T, embed_table.T, embed_table.T,
      jnp.eye(128, dtype=jnp.float32))

    mv4 = _make_sc_gather(B, b_per_w, info.num_cores)(table4, idx4)

    W3s = jnp.tile(W3[:D], (PACK, 1))
    W3b = W3[D:]

    BLK = 4096
    grid = (B // BLK,)

    cvt = pl.pallas_call(
        _content_body,
        grid=grid,
        in_specs=[
            pl.BlockSpec((BLK, NC_FEAT), lambda i: (i, 0)),
            pl.BlockSpec((NC_FEAT, H1), lambda i: (0, 0)),
            pl.BlockSpec((1, H1), lambda i: (0, 0)),
            pl.BlockSpec((H1, H2), lambda i: (0, 0)),
            pl.BlockSpec((H2, 1), lambda i: (0, 0)),
        ],
        out_specs=pl.BlockSpec((H2, BLK), lambda i: (0, i)),
        out_shape=jax.ShapeDtypeStruct((H2, B), jnp.float32),
    )(content_features, W1, b1.reshape(1, H1), W2, b2.reshape(H2, 1))

    out = pl.pallas_call(
        _final_body,
        grid=grid,
        in_specs=[
            pl.BlockSpec((BLK, 128), lambda i: (i, 0)),
            pl.BlockSpec((BLK, 1), lambda i: (i, 0)),
            pl.BlockSpec((H2, BLK), lambda i: (0, i)),
            pl.BlockSpec((128, H3), lambda i: (0, 0)),
            pl.BlockSpec((H2, H3), lambda i: (0, 0)),
            pl.BlockSpec((1, H3), lambda i: (0, 0)),
            pl.BlockSpec((H3, OUT), lambda i: (0, 0)),
            pl.BlockSpec((OUT, 1), lambda i: (0, 0)),
        ],
        out_specs=pl.BlockSpec((OUT, BLK), lambda i: (0, i)),
        out_shape=jax.ShapeDtypeStruct((OUT, B), jnp.float32),
    )(mv4, off, cvt, W3s, W3b, b3.reshape(1, H3), W4, b4.reshape(OUT, 1))
    return out.T


# speedup vs baseline: 1.0018x; 1.0018x over previous
import functools

import jax
import jax.numpy as jnp
from jax import lax
from jax.experimental import pallas as pl
from jax.experimental.pallas import tpu as pltpu
from jax.experimental.pallas import tpu_sc as plsc

_LBLK = 8192


def _transpose_pack_body(x0_ref, x1_ref, x2_ref, x3_ref, eye_ref, out_ref):
    X = jnp.concatenate(
        [x0_ref[...], x1_ref[...], x2_ref[...], x3_ref[...]], axis=0)
    out_ref[...] = lax.dot_general(
        X, eye_ref[...], (((0,), (0,)), ((), ())),
        preferred_element_type=jnp.float32)


def _make_sc_gather(B, b_per_w, NC):
    mesh = plsc.VectorSubcoreMesh(core_axis_name="c", subcore_axis_name="s")

    @functools.partial(
        pl.kernel,
        mesh=mesh,
        compiler_params=pltpu.CompilerParams(
            needs_layout_passes=False, use_tc_tiling_on_sc=True),
        out_type=jax.ShapeDtypeStruct((B, 128), jnp.float32),
        scratch_types=[
            pltpu.VMEM((b_per_w,), jnp.int32),
            pltpu.VMEM((b_per_w, 128), jnp.float32),
            pltpu.SemaphoreType.DMA,
        ],
    )
    def sc_gather(table_hbm, idx_hbm, out_hbm, idx_v, rows_v, sem):
        wid = lax.axis_index("s") * NC + lax.axis_index("c")
        base = wid * b_per_w
        pltpu.sync_copy(idx_hbm.at[pl.ds(base, b_per_w)], idx_v)
        pltpu.async_copy(table_hbm.at[idx_v], rows_v, sem).wait()
        pltpu.sync_copy(rows_v, out_hbm.at[pl.ds(base, b_per_w)])

    return sc_gather


def _content_body(x_ref, w1_ref, b1_ref, w2_ref, b2_ref, cvt_ref):
    h = jnp.maximum(
        jnp.dot(x_ref[...], w1_ref[...], preferred_element_type=jnp.float32)
        + b1_ref[...], 0.0)
    cvt_ref[...] = (
        lax.dot_general(w2_ref[...], h, (((0,), (1,)), ((), ())),
                        preferred_element_type=jnp.float32)
        + b2_ref[...])


def _final_body(mv4_ref, off_ref, cvt_ref, w3s_ref, w3b_ref, b3_ref,
                w4_ref, b4_ref, out_ref):
    seg = lax.broadcasted_iota(jnp.int32, mv4_ref.shape, 1) // 32
    mv4m = jnp.where(seg == off_ref[...], mv4_ref[...], 0.0)
    h2 = jnp.maximum(
        jnp.dot(mv4m, w3s_ref[...], preferred_element_type=jnp.float32)
        + lax.dot_general(cvt_ref[...], w3b_ref[...], (((0,), (0,)), ((), ())),
                          preferred_element_type=jnp.float32)
        + b3_ref[...], 0.0)
    out_ref[...] = (
        lax.dot_general(w4_ref[...], h2, (((0,), (1,)), ((), ())),
                        preferred_element_type=jnp.float32)
        + b4_ref[...])


def kernel(movie_ids, content_features, embed_table, W1, b1, W2, b2, W3, b3, W4, b4):
    B, NC_FEAT = content_features.shape
    V, D = embed_table.shape
    H1 = W1.shape[1]
    H2 = W2.shape[1]
    H3 = W3.shape[1]
    OUT = W4.shape[1]
    PACK = 128 // D

    info = plsc.get_sparse_core_info()
    NW = info.num_cores * info.num_subcores
    b_per_w = B // NW

    rblk = 8192
    nblk = -(-(V // PACK) // rblk)
    V4 = nblk * rblk
    ids = movie_ids.astype(jnp.int32)
    idx4 = ids % V4
    off = (ids // V4).reshape(B, 1)
    xmax = (V - 1) // rblk

    def _xmap(a):
        return lambda i: (0, jnp.minimum(i + a * nblk, xmax))

    table4 = pl.pallas_call(
        _transpose_pack_body,
        grid=(nblk,),
        in_specs=[
            pl.BlockSpec((D, rblk), _xmap(0)),
            pl.BlockSpec((D, rblk), _xmap(1)),
            pl.BlockSpec((D, rblk), _xmap(2)),
            pl.BlockSpec((D, rblk), _xmap(3)),
            pl.BlockSpec((128, 128), lambda i: (0, 0)),
        ],
        out_specs=pl.BlockSpec((rblk, 128), lambda i: (i, 0)),
        out_shape=jax.ShapeDtypeStruct((V4, 128), jnp.float32),
    )(embed_table.T, embed_table.T, embed_table.T, embed_table.T,
      jnp.eye(128, dtype=jnp.float32))

    BLK = 4096
    grid = (B // BLK,)

    cvt = pl.pallas_call(
        _content_body,
        grid=grid,
        in_specs=[
            pl.BlockSpec((BLK, NC_FEAT), lambda i: (i, 0)),
            pl.BlockSpec((NC_FEAT, H1), lambda i: (0, 0)),
            pl.BlockSpec((1, H1), lambda i: (0, 0)),
            pl.BlockSpec((H1, H2), lambda i: (0, 0)),
            pl.BlockSpec((H2, 1), lambda i: (0, 0)),
        ],
        out_specs=pl.BlockSpec((H2, BLK), lambda i: (0, i)),
        out_shape=jax.ShapeDtypeStruct((H2, B), jnp.float32),
    )(content_features, W1, b1.reshape(1, H1), W2, b2.reshape(H2, 1))

    mv4 = _make_sc_gather(B, b_per_w, info.num_cores)(table4, idx4)

    W3s = jnp.tile(W3[:D], (PACK, 1))
    W3b = W3[D:]

    out = pl.pallas_call(
        _final_body,
        grid=grid,
        in_specs=[
            pl.BlockSpec((BLK, 128), lambda i: (i, 0)),
            pl.BlockSpec((BLK, 1), lambda i: (i, 0)),
            pl.BlockSpec((H2, BLK), lambda i: (0, i)),
            pl.BlockSpec((128, H3), lambda i: (0, 0)),
            pl.BlockSpec((H2, H3), lambda i: (0, 0)),
            pl.BlockSpec((1, H3), lambda i: (0, 0)),
            pl.BlockSpec((H3, OUT), lambda i: (0, 0)),
            pl.BlockSpec((OUT, 1), lambda i: (0, 0)),
        ],
        out_specs=pl.BlockSpec((OUT, BLK), lambda i: (0, i)),
        out_shape=jax.ShapeDtypeStruct((OUT, B), jnp.float32),
    )(mv4, off, cvt, W3s, W3b, b3.reshape(1, H3), W4, b4.reshape(OUT, 1))
    return out.T


# speedup vs baseline: 1.0540x; 1.0521x over previous
import functools

import jax
import jax.numpy as jnp
from jax import lax
from jax.experimental import pallas as pl
from jax.experimental.pallas import tpu as pltpu
from jax.experimental.pallas import tpu_sc as plsc

_LBLK = 8192


def _transpose_pack_body(x0_ref, x1_ref, x2_ref, x3_ref, eye_ref, out_ref):
    X = jnp.concatenate(
        [x0_ref[...], x1_ref[...], x2_ref[...], x3_ref[...]], axis=0)
    out_ref[...] = lax.dot_general(
        X, eye_ref[...], (((0,), (0,)), ((), ())),
        preferred_element_type=jnp.float32)


def _make_sc_gather(B, b_per_w, NC):
    mesh = plsc.VectorSubcoreMesh(core_axis_name="c", subcore_axis_name="s")

    @functools.partial(
        pl.kernel,
        mesh=mesh,
        compiler_params=pltpu.CompilerParams(
            needs_layout_passes=False, use_tc_tiling_on_sc=True),
        out_type=jax.ShapeDtypeStruct((B, 128), jnp.float32),
        scratch_types=[
            pltpu.VMEM((b_per_w,), jnp.int32),
            pltpu.VMEM((b_per_w, 128), jnp.float32),
            pltpu.SemaphoreType.DMA,
        ],
    )
    def sc_gather(table_hbm, idx_hbm, out_hbm, idx_v, rows_v, sem):
        wid = lax.axis_index("s") * NC + lax.axis_index("c")
        base = wid * b_per_w
        pltpu.sync_copy(idx_hbm.at[pl.ds(base, b_per_w)], idx_v)
        pltpu.async_copy(table_hbm.at[idx_v], rows_v, sem).wait()
        pltpu.sync_copy(rows_v, out_hbm.at[pl.ds(base, b_per_w)])

    return sc_gather


def _tower_body(x_ref, mv4_ref, off_ref, w1_ref, b1_ref, w2_ref, b2_ref,
                w3s_ref, w3b_ref, b3_ref, w4_ref, b4_ref, out_ref):
    h = jnp.maximum(
        jnp.dot(x_ref[...], w1_ref[...], preferred_element_type=jnp.float32)
        + b1_ref[...], 0.0)
    cv = jnp.dot(h, w2_ref[...], preferred_element_type=jnp.float32) + b2_ref[...]
    seg = lax.broadcasted_iota(jnp.int32, mv4_ref.shape, 1) // 32
    mv4m = jnp.where(seg == off_ref[...], mv4_ref[...], 0.0)
    h2 = jnp.maximum(
        jnp.dot(mv4m, w3s_ref[...], preferred_element_type=jnp.float32)
        + jnp.dot(cv, w3b_ref[...], preferred_element_type=jnp.float32)
        + b3_ref[...], 0.0)
    out_ref[...] = (
        lax.dot_general(w4_ref[...], h2, (((0,), (1,)), ((), ())),
                        preferred_element_type=jnp.float32)
        + b4_ref[...])


def kernel(movie_ids, content_features, embed_table, W1, b1, W2, b2, W3, b3, W4, b4):
    B, NC_FEAT = content_features.shape
    V, D = embed_table.shape
    H1 = W1.shape[1]
    H2 = W2.shape[1]
    H3 = W3.shape[1]
    OUT = W4.shape[1]
    PACK = 128 // D

    info = plsc.get_sparse_core_info()
    NW = info.num_cores * info.num_subcores
    b_per_w = B // NW

    rblk = 8192
    nblk = -(-(V // PACK) // rblk)
    V4 = nblk * rblk
    ids = movie_ids.astype(jnp.int32)
    idx4 = ids % V4
    off = (ids // V4).reshape(B, 1)
    xmax = (V - 1) // rblk

    def _xmap(a):
        return lambda i: (0, jnp.minimum(i + a * nblk, xmax))

    table4 = pl.pallas_call(
        _transpose_pack_body,
        grid=(nblk,),
        in_specs=[
            pl.BlockSpec((D, rblk), _xmap(0)),
            pl.BlockSpec((D, rblk), _xmap(1)),
            pl.BlockSpec((D, rblk), _xmap(2)),
            pl.BlockSpec((D, rblk), _xmap(3)),
            pl.BlockSpec((128, 128), lambda i: (0, 0)),
        ],
        out_specs=pl.BlockSpec((rblk, 128), lambda i: (i, 0)),
        out_shape=jax.ShapeDtypeStruct((V4, 128), jnp.float32),
    )(embed_table.T, embed_table.T, embed_table.T, embed_table.T,
      jnp.eye(128, dtype=jnp.float32))

    mv4 = _make_sc_gather(B, b_per_w, info.num_cores)(table4, idx4)

    W3s = jnp.tile(W3[:D], (PACK, 1))
    W3b = W3[D:]

    BLK = 4096
    grid = (B // BLK,)

    out = pl.pallas_call(
        _tower_body,
        grid=grid,
        in_specs=[
            pl.BlockSpec((BLK, NC_FEAT), lambda i: (i, 0)),
            pl.BlockSpec((BLK, 128), lambda i: (i, 0)),
            pl.BlockSpec((BLK, 1), lambda i: (i, 0)),
            pl.BlockSpec((NC_FEAT, H1), lambda i: (0, 0)),
            pl.BlockSpec((1, H1), lambda i: (0, 0)),
            pl.BlockSpec((H1, H2), lambda i: (0, 0)),
            pl.BlockSpec((1, H2), lambda i: (0, 0)),
            pl.BlockSpec((128, H3), lambda i: (0, 0)),
            pl.BlockSpec((H2, H3), lambda i: (0, 0)),
            pl.BlockSpec((1, H3), lambda i: (0, 0)),
            pl.BlockSpec((H3, OUT), lambda i: (0, 0)),
            pl.BlockSpec((OUT, 1), lambda i: (0, 0)),
        ],
        out_specs=pl.BlockSpec((OUT, BLK), lambda i: (0, i)),
        out_shape=jax.ShapeDtypeStruct((OUT, B), jnp.float32),
    )(content_features, mv4, off, W1, b1.reshape(1, H1), W2, b2.reshape(1, H2),
      W3s, W3b, b3.reshape(1, H3), W4, b4.reshape(OUT, 1))
    return out.T
